# baseline (speedup 1.0000x reference)
import functools

import jax
import jax.numpy as jnp
from jax import lax
from jax.experimental import pallas as pl
from jax.experimental.pallas import tpu as pltpu
from jax.experimental.pallas import tpu_sc as plsc

VOCAB_ROWS = 32
EMBED_DIM = 64
BATCH = 4096
SEQ = 200
TOTAL = BATCH * SEQ

_info = plsc.get_sparse_core_info()
_NC = _info.num_cores
_NS = _info.num_subcores
_NW = _NC * _NS
PER_W = TOTAL // _NW
CHUNK = 128
N_CHUNKS = PER_W // CHUNK
NBUF = 4
SKEW = 2


def _make_kernel():
    mesh = plsc.VectorSubcoreMesh(core_axis_name="c", subcore_axis_name="s")

    @functools.partial(
        pl.kernel,
        mesh=mesh,
        out_type=jax.ShapeDtypeStruct((TOTAL, EMBED_DIM), jnp.float32),
        compiler_params=pltpu.CompilerParams(use_tc_tiling_on_sc=False),
        scratch_types=[
            pltpu.VMEM((PER_W,), jnp.int32),
            pltpu.VMEM((NBUF, CHUNK, EMBED_DIM), jnp.float32),
        ]
        + [pltpu.SemaphoreType.DMA] * (2 * NBUF),
    )
    def k(idx_hbm, table_hbm, out_hbm, idx_v, rows,
          g0, g1, g2, g3, o0, o1, o2, o3):
        gsem = [g0, g1, g2, g3]
        osem = [o0, o1, o2, o3]
        wid = lax.axis_index("s") * _NC + lax.axis_index("c")
        base = wid * PER_W

        pltpu.sync_copy(idx_hbm.at[pl.ds(base, PER_W)], idx_v)

        def sg(q, b, start):
            cp = pltpu.make_async_copy(
                table_hbm.at[idx_v.at[pl.ds(q * CHUNK, CHUNK)]],
                rows.at[b], gsem[b])
            cp.start() if start else cp.wait()

        def sw(q, b, start):
            cp = pltpu.make_async_copy(
                rows.at[b],
                out_hbm.at[pl.ds(base + q * CHUNK, CHUNK)], osem[b])
            cp.start() if start else cp.wait()

        for b in range(NBUF):
            sg(b, b, True)
        for b in range(SKEW):
            sg(b, b, False)
            sw(b, b, True)

        def body(i, carry):
            qb = i * NBUF
            for b in range(NBUF):
                q = qb + b
                sw(q - NBUF, b, False)
                sg(q, b, True)
                qw = q - SKEW
                bw = (b + NBUF - SKEW) % NBUF
                sg(qw, bw, False)
                sw(qw, bw, True)
            return carry

        lax.fori_loop(1, N_CHUNKS // NBUF, body, 0)

        lastq = N_CHUNKS - NBUF
        for b in range(SKEW, NBUF):
            sg(lastq + b, b, False)
            sw(lastq + b, b, True)
        for b in range(NBUF):
            sw(lastq + b, b, False)

    return k


_sc_gather = _make_kernel()


def kernel(inputs, embedding_table):
    idx = inputs.reshape(TOTAL)
    out = _sc_gather(idx, embedding_table)
    return out.reshape(BATCH, SEQ, EMBED_DIM)


# speedup vs baseline: 3.0618x; 3.0618x over previous
import functools

import jax
import jax.numpy as jnp
from jax import lax
from jax.experimental import pallas as pl
from jax.experimental.pallas import tpu as pltpu
from jax.experimental.pallas import tpu_sc as plsc

VOCAB_ROWS = 32
EMBED_DIM = 64
BATCH = 4096
SEQ = 200
TOTAL = BATCH * SEQ

_info = plsc.get_sparse_core_info()
_NC = _info.num_cores
_NS = _info.num_subcores
_NW = _NC * _NS
PER_W = TOTAL // _NW
CHUNK = 256
N_CHUNKS = PER_W // CHUNK
NBUF = 4
SKEW = 2


def _make_kernel():
    mesh = plsc.VectorSubcoreMesh(core_axis_name="c", subcore_axis_name="s")

    @functools.partial(
        pl.kernel,
        mesh=mesh,
        out_type=jax.ShapeDtypeStruct((TOTAL, EMBED_DIM), jnp.float32),
        compiler_params=pltpu.CompilerParams(use_tc_tiling_on_sc=False),
        scratch_types=[
            pltpu.VMEM((PER_W,), jnp.int32),
            pltpu.VMEM((NBUF, CHUNK, EMBED_DIM), jnp.float32),
            pltpu.VMEM_SHARED((VOCAB_ROWS, EMBED_DIM), jnp.float32),
        ]
        + [pltpu.SemaphoreType.DMA] * (2 * NBUF),
    )
    def k(idx_hbm, table_hbm, out_hbm, idx_v, rows, table_sh,
          g0, g1, g2, g3, o0, o1, o2, o3):
        gsem = [g0, g1, g2, g3]
        osem = [o0, o1, o2, o3]
        wid = lax.axis_index("s") * _NC + lax.axis_index("c")
        base = wid * PER_W

        @pl.when(lax.axis_index("s") == 0)
        def _():
            pltpu.sync_copy(table_hbm, table_sh)

        pltpu.sync_copy(idx_hbm.at[pl.ds(base, PER_W)], idx_v)
        plsc.subcore_barrier()

        def sg(q, b, start):
            cp = pltpu.make_async_copy(
                table_sh.at[idx_v.at[pl.ds(q * CHUNK, CHUNK)]],
                rows.at[b], gsem[b])
            cp.start() if start else cp.wait()

        def sw(q, b, start):
            cp = pltpu.make_async_copy(
                rows.at[b],
                out_hbm.at[pl.ds(base + q * CHUNK, CHUNK)], osem[b])
            cp.start() if start else cp.wait()

        for b in range(NBUF):
            sg(b, b, True)
        for b in range(SKEW):
            sg(b, b, False)
            sw(b, b, True)

        def body(i, carry):
            qb = i * NBUF
            for b in range(NBUF):
                q = qb + b
                sw(q - NBUF, b, False)
                sg(q, b, True)
                qw = q - SKEW
                bw = (b + NBUF - SKEW) % NBUF
                sg(qw, bw, False)
                sw(qw, bw, True)
            return carry

        lax.fori_loop(1, N_CHUNKS // NBUF, body, 0)

        lastq = N_CHUNKS - NBUF
        for b in range(SKEW, NBUF):
            sg(lastq + b, b, False)
            sw(lastq + b, b, True)
        for b in range(NBUF):
            sw(lastq + b, b, False)

    return k


_sc_gather = _make_kernel()


def kernel(inputs, embedding_table):
    idx = inputs.reshape(TOTAL)
    out = _sc_gather(idx, embedding_table)
    return out.reshape(BATCH, SEQ, EMBED_DIM)


# speedup vs baseline: 3.0689x; 1.0023x over previous
import functools

import jax
import jax.numpy as jnp
from jax import lax
from jax.experimental import pallas as pl
from jax.experimental.pallas import tpu as pltpu
from jax.experimental.pallas import tpu_sc as plsc

VOCAB_ROWS = 32
EMBED_DIM = 64
BATCH = 4096
SEQ = 200
TOTAL = BATCH * SEQ
PAIR_ROWS = VOCAB_ROWS * VOCAB_ROWS
PAIR_DIM = 2 * EMBED_DIM
TOTAL2 = TOTAL // 2

_info = plsc.get_sparse_core_info()
_NC = _info.num_cores
_NS = _info.num_subcores
_NW = _NC * _NS
_L = _info.num_lanes
PER_W = TOTAL // _NW
PAIR_W = PER_W // 2
CHUNK = 128
N_CHUNKS = PAIR_W // CHUNK
NBUF = 4
SKEW = 2


def _make_kernel():
    mesh = plsc.VectorSubcoreMesh(core_axis_name="c", subcore_axis_name="s")

    @functools.partial(
        pl.kernel,
        mesh=mesh,
        out_type=jax.ShapeDtypeStruct((TOTAL2, PAIR_DIM), jnp.float32),
        compiler_params=pltpu.CompilerParams(
            use_tc_tiling_on_sc=False, needs_layout_passes=False),
        scratch_types=[
            pltpu.VMEM((PER_W,), jnp.int32),
            pltpu.VMEM((PAIR_W,), jnp.int32),
            pltpu.VMEM((NBUF, CHUNK, PAIR_DIM), jnp.float32),
            pltpu.VMEM_SHARED((PAIR_ROWS, PAIR_DIM), jnp.float32),
        ]
        + [pltpu.SemaphoreType.DMA] * (2 * NBUF),
    )
    def k(idx_hbm, ptable_hbm, out_hbm, idx_v, idx2_v, rows, ptable_sh,
          g0, g1, g2, g3, o0, o1, o2, o3):
        gsem = [g0, g1, g2, g3]
        osem = [o0, o1, o2, o3]
        sid = lax.axis_index("s")
        wid = sid * _NC + lax.axis_index("c")
        base = wid * PER_W
        base2 = wid * PAIR_W

        @pl.when(sid == 0)
        def _():
            pltpu.sync_copy(ptable_hbm, ptable_sh)

        pltpu.sync_copy(idx_hbm.at[pl.ds(base, PER_W)], idx_v)

        lanes = lax.iota(jnp.int32, _L)

        @plsc.parallel_loop(0, PAIR_W // _L, unroll=4)
        def pack(g):
            pos = (g * _L + lanes) * 2
            a = plsc.load_gather(idx_v, [pos])
            b = plsc.load_gather(idx_v, [pos + 1])
            idx2_v[pl.ds(g * _L, _L)] = (a << 5) + b

        plsc.subcore_barrier()

        def sg(q, b, start):
            cp = pltpu.make_async_copy(
                ptable_sh.at[idx2_v.at[pl.ds(q * CHUNK, CHUNK)]],
                rows.at[b], gsem[b])
            cp.start() if start else cp.wait()

        def sw(q, b, start):
            cp = pltpu.make_async_copy(
                rows.at[b],
                out_hbm.at[pl.ds(base2 + q * CHUNK, CHUNK)], osem[b])
            cp.start() if start else cp.wait()

        for b in range(NBUF):
            sg(b, b, True)
        for b in range(SKEW):
            sg(b, b, False)
            sw(b, b, True)

        def body(i, carry):
            qb = i * NBUF
            for b in range(NBUF):
                q = qb + b
                sw(q - NBUF, b, False)
                sg(q, b, True)
                qw = q - SKEW
                bw = (b + NBUF - SKEW) % NBUF
                sg(qw, bw, False)
                sw(qw, bw, True)
            return carry

        lax.fori_loop(1, N_CHUNKS // NBUF, body, 0)

        lastq = N_CHUNKS - NBUF
        for b in range(SKEW, NBUF):
            sg(lastq + b, b, False)
            sw(lastq + b, b, True)
        for b in range(NBUF):
            sw(lastq + b, b, False)

    return k


_sc_gather = _make_kernel()


def kernel(inputs, embedding_table):
    idx = inputs.reshape(TOTAL)
    left = jnp.broadcast_to(embedding_table[:, None, :],
                            (VOCAB_ROWS, VOCAB_ROWS, EMBED_DIM))
    right = jnp.broadcast_to(embedding_table[None, :, :],
                             (VOCAB_ROWS, VOCAB_ROWS, EMBED_DIM))
    ptable = jnp.concatenate([left, right], axis=-1).reshape(
        PAIR_ROWS, PAIR_DIM)
    out = _sc_gather(idx, ptable)
    return out.reshape(BATCH, SEQ, EMBED_DIM)
